# Initial kernel scaffold; baseline (speedup 1.0000x reference)
import jax
import jax.numpy as jnp
from jax.experimental import pallas as pl


def kernel(pred, target):
    raise NotImplementedError("write your pallas kernel here")


# speedup vs baseline: 3.6495x; 3.6495x over previous
import functools
import math

import jax
import jax.numpy as jnp
from jax import lax
from jax.experimental import pallas as pl
from jax.experimental.pallas import tpu as pltpu
from jax.experimental.pallas import tpu_sc as plsc

B = 16384
C = 1000
N_BINS = 15
VIRTUAL_PROB = 0.9

RBLK = 256

NSUB = 16
LANES = 16
RW = B // NSUB
NV = RW // LANES

_Q = (1.0 - VIRTUAL_PROB) / (C - 1)
_KL_CONST = VIRTUAL_PROB * math.log(VIRTUAL_PROB) + (C - 1) * _Q * math.log(_Q)
_PMQ = VIRTUAL_PROB - _Q


def _tc_stats_body(x_ref, tgt_ref, conf_ref, acc_ref, ent_ref, kl_ref):
    x = x_ref[...]
    tgt = tgt_ref[...]
    m = jnp.max(x, axis=1)
    e = jnp.exp(x - m[:, None])
    s = jnp.sum(e, axis=1)
    dot = jnp.sum(e * x, axis=1)
    sum_pred = jnp.sum(x, axis=1)
    cols = lax.broadcasted_iota(jnp.int32, (RBLK, C), 1)
    t_logit = jnp.sum(jnp.where(cols == tgt[:, None], x, 0.0), axis=1)
    lse = m + jnp.log(s)
    conf_ref[...] = 1.0 / s
    acc_ref[...] = (t_logit == m).astype(jnp.float32)
    ent_ref[...] = lse - dot / s
    kl_ref[...] = (jnp.float32(_KL_CONST)
                   - jnp.float32(_Q) * (sum_pred - jnp.float32(C) * lse)
                   - jnp.float32(_PMQ) * (t_logit - lse))


def _tc_stats(pred, target):
    grid = (B // RBLK,)
    row_spec = pl.BlockSpec((RBLK,), lambda i: (i,))
    return pl.pallas_call(
        _tc_stats_body,
        grid=grid,
        in_specs=[
            pl.BlockSpec((RBLK, C), lambda i: (i, 0)),
            row_spec,
        ],
        out_specs=[row_spec, row_spec, row_spec, row_spec],
        out_shape=[jax.ShapeDtypeStruct((B,), jnp.float32)] * 4,
    )(pred, target)


def _sc_finish_body(conf_hbm, acc_hbm, ent_hbm, kl_hbm, bnd_hbm, out_hbm,
                    conf_v, acc_v, ent_v, kl_v, bnd_v, hist_v, acc2_v,
                    tmp_v, shared, out_v):
    sid = lax.axis_index("s")
    base = sid * RW
    pltpu.sync_copy(conf_hbm.at[pl.ds(base, RW)], conf_v)
    pltpu.sync_copy(acc_hbm.at[pl.ds(base, RW)], acc_v)
    pltpu.sync_copy(ent_hbm.at[pl.ds(base, RW)], ent_v)
    pltpu.sync_copy(kl_hbm.at[pl.ds(base, RW)], kl_v)
    pltpu.sync_copy(bnd_hbm, bnd_v)

    lane = lax.iota(jnp.int32, LANES)
    zero16 = jnp.zeros((LANES,), jnp.float32)
    ones16 = jnp.ones((LANES,), jnp.float32)

    for r in range(48):
        hist_v[r, :] = zero16
        acc2_v[r, :] = zero16

    def vbody(j, carry):
        aa, ea, ka = carry
        sl = pl.ds(j * LANES, LANES)
        cv = conf_v[sl]
        av = acc_v[sl]
        ev = ent_v[sl]
        kv = kl_v[sl]
        idx = jnp.zeros((LANES,), jnp.int32)
        for i in range(N_BINS - 1):
            idx = idx + (cv > bnd_v[i, :]).astype(jnp.int32)
        plsc.addupdate_scatter(hist_v, [idx, lane], ones16)
        plsc.addupdate_scatter(hist_v, [idx + N_BINS, lane], cv)
        plsc.addupdate_scatter(hist_v, [idx + 2 * N_BINS, lane], av)
        return (aa + av, ea + ev, ka + kv)

    aa, ea, ka = lax.fori_loop(0, NV, vbody, (zero16, zero16, zero16))
    hist_v[45, :] = aa
    hist_v[46, :] = ea
    hist_v[47, :] = ka

    pltpu.sync_copy(hist_v, shared.at[sid])
    plsc.subcore_barrier()

    for w in range(NSUB):
        pltpu.sync_copy(shared.at[w], tmp_v)
        for r in range(48):
            acc2_v[r, :] = acc2_v[r, :] + tmp_v[r, :]

    inv_b = 1.0 / B
    dv = zero16
    for b in range(N_BINS):
        diff = (jnp.sum(acc2_v[N_BINS + b, :])
                - jnp.sum(acc2_v[2 * N_BINS + b, :]))
        dv = jnp.where(lane == b, diff, dv)
    ece = jnp.sum(jnp.abs(dv)) * inv_b
    lane0 = lane == 0
    out_v[0, :] = jnp.where(lane0, jnp.sum(acc2_v[45, :]) * inv_b, 0.0)
    out_v[1, :] = jnp.where(lane0, ece, 0.0)
    out_v[2, :] = jnp.where(lane0, jnp.sum(acc2_v[46, :]) * inv_b, 0.0)
    out_v[3, :] = jnp.where(lane0, jnp.sum(acc2_v[47, :]) * inv_b, 0.0)
    pltpu.sync_copy(out_v, out_hbm)


def _sc_finish(conf, acc, ent, kl, bounds):
    mesh = plsc.VectorSubcoreMesh(core_axis_name="c", subcore_axis_name="s",
                                  num_cores=1)
    f = functools.partial(
        pl.kernel,
        out_type=jax.ShapeDtypeStruct((4, LANES), jnp.float32),
        mesh=mesh,
        compiler_params=pltpu.CompilerParams(needs_layout_passes=False),
        scratch_types=[
            pltpu.VMEM((RW,), jnp.float32),
            pltpu.VMEM((RW,), jnp.float32),
            pltpu.VMEM((RW,), jnp.float32),
            pltpu.VMEM((RW,), jnp.float32),
            pltpu.VMEM((N_BINS - 1, LANES), jnp.float32),
            pltpu.VMEM((48, LANES), jnp.float32),
            pltpu.VMEM((48, LANES), jnp.float32),
            pltpu.VMEM((48, LANES), jnp.float32),
            pltpu.VMEM_SHARED((NSUB, 48, LANES), jnp.float32),
            pltpu.VMEM((4, LANES), jnp.float32),
        ],
    )(_sc_finish_body)
    return f(conf, acc, ent, kl, bounds)


def kernel(pred, target):
    conf, acc, ent, kl = _tc_stats(pred, target)
    boundaries = jnp.linspace(0.0, 1.0, N_BINS + 1)
    bounds = jnp.broadcast_to(boundaries[1:N_BINS, None],
                              (N_BINS - 1, LANES)).astype(jnp.float32)
    out = _sc_finish(conf, acc, ent, kl, bounds)
    acc1 = out[0, 0:1]
    ece = out[1, 0:1]
    entropy = out[2, 0]
    kl_div = out[3, 0]
    return (acc1, ece, entropy, kl_div)


# speedup vs baseline: 4.7612x; 1.3046x over previous
import functools
import math

import jax
import jax.numpy as jnp
from jax import lax
from jax.experimental import pallas as pl
from jax.experimental.pallas import tpu as pltpu
from jax.experimental.pallas import tpu_sc as plsc

B = 16384
C = 1000
N_BINS = 15
VIRTUAL_PROB = 0.9

RBLK = 256
NBLK = B // RBLK

_Q = (1.0 - VIRTUAL_PROB) / (C - 1)
_KL_CONST = VIRTUAL_PROB * math.log(VIRTUAL_PROB) + (C - 1) * _Q * math.log(_Q)
_PMQ = VIRTUAL_PROB - _Q


def _tc_body(x_ref, tgt_ref, bnd_ref, acc1_ref, ece_ref, ent_ref, kl_ref,
             hist_ref):
    pid = pl.program_id(0)
    x = x_ref[...]
    tgt = tgt_ref[...]
    m = jnp.max(x, axis=1)
    e = jnp.exp(x - m[:, None])
    s = jnp.sum(e, axis=1)
    dot = jnp.sum(e * x, axis=1)
    sum_pred = jnp.sum(x, axis=1)
    cols = lax.broadcasted_iota(jnp.int32, (RBLK, C), 1)
    t_logit = jnp.sum(jnp.where(cols == tgt[:, None], x, 0.0), axis=1)
    lse = m + jnp.log(s)
    conf = 1.0 / s
    acc = (t_logit == m).astype(jnp.float32)
    ent_row = lse - dot / s
    kl_row = (jnp.float32(_KL_CONST)
              - jnp.float32(_Q) * (sum_pred - jnp.float32(C) * lse)
              - jnp.float32(_PMQ) * (t_logit - lse))

    cmp = (conf[:, None] > bnd_ref[...]).astype(jnp.int32)
    idx = jnp.sum(cmp[:, 1:N_BINS], axis=1)

    lanes = lax.broadcasted_iota(jnp.int32, (1, 128), 1)
    onehot = idx[:, None] == lanes
    diff = conf - acc
    contrib = jnp.sum(jnp.where(onehot, diff[:, None], 0.0), axis=0)
    scal = jnp.where(lanes[0] == 15, jnp.sum(acc),
                     jnp.where(lanes[0] == 16, jnp.sum(ent_row),
                               jnp.where(lanes[0] == 17, jnp.sum(kl_row),
                                         0.0)))
    upd = (contrib + scal)[None, :]

    @pl.when(pid == 0)
    def _():
        hist_ref[...] = upd

    @pl.when(pid > 0)
    def _():
        hist_ref[...] = hist_ref[...] + upd

    @pl.when(pid == NBLK - 1)
    def _():
        inv_b = 1.0 / B
        h = hist_ref[0, :]
        lane = lanes[0]
        ece = jnp.sum(jnp.where(lane < N_BINS, jnp.abs(h), 0.0)) * inv_b
        acc1 = jnp.sum(jnp.where(lane == 15, h, 0.0)) * inv_b
        ent = jnp.sum(jnp.where(lane == 16, h, 0.0)) * inv_b
        kl = jnp.sum(jnp.where(lane == 17, h, 0.0)) * inv_b
        acc1_ref[...] = acc1.reshape(1, 1)
        ece_ref[...] = ece.reshape(1, 1)
        ent_ref[...] = ent.reshape(1, 1)
        kl_ref[...] = kl.reshape(1, 1)


def _tc_metrics(pred, target, bounds):
    one_spec = pl.BlockSpec((1, 1), lambda i: (0, 0))
    outs = pl.pallas_call(
        _tc_body,
        grid=(NBLK,),
        in_specs=[
            pl.BlockSpec((RBLK, C), lambda i: (i, 0)),
            pl.BlockSpec((RBLK,), lambda i: (i,)),
            pl.BlockSpec((1, N_BINS + 1), lambda i: (0, 0)),
        ],
        out_specs=[one_spec, one_spec, one_spec, one_spec],
        out_shape=[jax.ShapeDtypeStruct((1, 1), jnp.float32)] * 4,
        scratch_shapes=[pltpu.VMEM((1, 128), jnp.float32)],
    )(pred, target, bounds)
    return outs


def kernel(pred, target):
    boundaries = jnp.linspace(0.0, 1.0, N_BINS + 1).astype(jnp.float32)
    bounds = boundaries.reshape(1, N_BINS + 1)
    acc1, ece, ent, kl = _tc_metrics(pred, target, bounds)
    return (acc1.reshape(1), ece.reshape(1), ent[0, 0], kl[0, 0])


# speedup vs baseline: 11.8515x; 2.4892x over previous
import math

import jax
import jax.numpy as jnp
from jax import lax
from jax.experimental import pallas as pl
from jax.experimental.pallas import tpu as pltpu

B = 16384
C = 1000
N_BINS = 15
VIRTUAL_PROB = 0.9

BBLK = 512
NBLK = B // BBLK

_Q = (1.0 - VIRTUAL_PROB) / (C - 1)
_KL_CONST = VIRTUAL_PROB * math.log(VIRTUAL_PROB) + (C - 1) * _Q * math.log(_Q)
_PMQ = VIRTUAL_PROB - _Q


def _tc_body(x_ref, tgt_ref, bnd_ref, acc1_ref, ece_ref, ent_ref, kl_ref,
             hist_ref):
    pid = pl.program_id(0)
    x = x_ref[...]
    tgt = tgt_ref[...]
    m = jnp.max(x, axis=0)
    e = jnp.exp(x - m[None, :])
    s = jnp.sum(e, axis=0)
    dot = jnp.sum(e * x, axis=0)
    sum_pred = jnp.sum(x, axis=0)
    rows = lax.broadcasted_iota(jnp.int32, (C, BBLK), 0)
    t_logit = jnp.sum(jnp.where(rows == tgt[None, :], x, 0.0), axis=0)
    lse = m + jnp.log(s)
    conf = 1.0 / s
    acc = (t_logit == m).astype(jnp.float32)
    ent_row = lse - dot / s
    kl_row = (jnp.float32(_KL_CONST)
              - jnp.float32(_Q) * (sum_pred - jnp.float32(C) * lse)
              - jnp.float32(_PMQ) * (t_logit - lse))

    cmp = (conf[:, None] > bnd_ref[...]).astype(jnp.int32)
    idx = jnp.sum(cmp[:, 1:N_BINS], axis=1)

    lanes = lax.broadcasted_iota(jnp.int32, (1, 128), 1)
    onehot = idx[:, None] == lanes
    diff = conf - acc
    contrib = jnp.sum(jnp.where(onehot, diff[:, None], 0.0), axis=0)
    scal = jnp.where(lanes[0] == 15, jnp.sum(acc),
                     jnp.where(lanes[0] == 16, jnp.sum(ent_row),
                               jnp.where(lanes[0] == 17, jnp.sum(kl_row),
                                         0.0)))
    upd = (contrib + scal)[None, :]

    @pl.when(pid == 0)
    def _():
        hist_ref[...] = upd

    @pl.when(pid > 0)
    def _():
        hist_ref[...] = hist_ref[...] + upd

    @pl.when(pid == NBLK - 1)
    def _():
        inv_b = 1.0 / B
        h = hist_ref[0, :]
        lane = lanes[0]
        ece = jnp.sum(jnp.where(lane < N_BINS, jnp.abs(h), 0.0)) * inv_b
        acc1 = jnp.sum(jnp.where(lane == 15, h, 0.0)) * inv_b
        ent = jnp.sum(jnp.where(lane == 16, h, 0.0)) * inv_b
        kl = jnp.sum(jnp.where(lane == 17, h, 0.0)) * inv_b
        acc1_ref[...] = acc1.reshape(1, 1)
        ece_ref[...] = ece.reshape(1, 1)
        ent_ref[...] = ent.reshape(1, 1)
        kl_ref[...] = kl.reshape(1, 1)


def _tc_metrics(pred_t, target, bounds):
    one_spec = pl.BlockSpec((1, 1), lambda i: (0, 0))
    return pl.pallas_call(
        _tc_body,
        grid=(NBLK,),
        in_specs=[
            pl.BlockSpec((C, BBLK), lambda i: (0, i)),
            pl.BlockSpec((BBLK,), lambda i: (i,)),
            pl.BlockSpec((1, N_BINS + 1), lambda i: (0, 0)),
        ],
        out_specs=[one_spec, one_spec, one_spec, one_spec],
        out_shape=[jax.ShapeDtypeStruct((1, 1), jnp.float32)] * 4,
        scratch_shapes=[pltpu.VMEM((1, 128), jnp.float32)],
    )(pred_t, target, bounds)


def kernel(pred, target):
    boundaries = jnp.linspace(0.0, 1.0, N_BINS + 1).astype(jnp.float32)
    bounds = boundaries.reshape(1, N_BINS + 1)
    acc1, ece, ent, kl = _tc_metrics(pred.T, target, bounds)
    return (acc1.reshape(1), ece.reshape(1), ent[0, 0], kl[0, 0])


# speedup vs baseline: 12.2067x; 1.0300x over previous
import math

import jax
import jax.numpy as jnp
from jax import lax
from jax.experimental import pallas as pl
from jax.experimental.pallas import tpu as pltpu

B = 16384
C = 1000
N_BINS = 15
VIRTUAL_PROB = 0.9

BBLK = 1024
NBLK = B // BBLK

_Q = (1.0 - VIRTUAL_PROB) / (C - 1)
_KL_CONST = VIRTUAL_PROB * math.log(VIRTUAL_PROB) + (C - 1) * _Q * math.log(_Q)
_PMQ = VIRTUAL_PROB - _Q


def _tc_body(x_ref, tgt_ref, bnd_ref, acc1_ref, ece_ref, ent_ref, kl_ref,
             hist_ref):
    pid = pl.program_id(0)
    x = x_ref[...]
    tgt = tgt_ref[...]
    m = jnp.max(x, axis=0)
    e = jnp.exp(x - m[None, :])
    s = jnp.sum(e, axis=0)
    dot = jnp.sum(e * x, axis=0)
    sum_pred = jnp.sum(x, axis=0)
    rows = lax.broadcasted_iota(jnp.int32, (C, BBLK), 0)
    t_logit = jnp.sum(jnp.where(rows == tgt[None, :], x, 0.0), axis=0)
    lse = m + jnp.log(s)
    conf = 1.0 / s
    acc = (t_logit == m).astype(jnp.float32)
    ent_row = lse - dot / s
    kl_row = (jnp.float32(_KL_CONST)
              - jnp.float32(_Q) * (sum_pred - jnp.float32(C) * lse)
              - jnp.float32(_PMQ) * (t_logit - lse))

    cmp = (conf[:, None] > bnd_ref[...]).astype(jnp.int32)
    idx = jnp.sum(cmp[:, 1:N_BINS], axis=1)

    lanes = lax.broadcasted_iota(jnp.int32, (1, 128), 1)
    onehot = idx[:, None] == lanes
    diff = conf - acc
    contrib = jnp.sum(jnp.where(onehot, diff[:, None], 0.0), axis=0)
    scal = jnp.where(lanes[0] == 15, jnp.sum(acc),
                     jnp.where(lanes[0] == 16, jnp.sum(ent_row),
                               jnp.where(lanes[0] == 17, jnp.sum(kl_row),
                                         0.0)))
    upd = (contrib + scal)[None, :]

    @pl.when(pid == 0)
    def _():
        hist_ref[...] = upd

    @pl.when(pid > 0)
    def _():
        hist_ref[...] = hist_ref[...] + upd

    @pl.when(pid == NBLK - 1)
    def _():
        inv_b = 1.0 / B
        h = hist_ref[0, :]
        lane = lanes[0]
        ece = jnp.sum(jnp.where(lane < N_BINS, jnp.abs(h), 0.0)) * inv_b
        acc1 = jnp.sum(jnp.where(lane == 15, h, 0.0)) * inv_b
        ent = jnp.sum(jnp.where(lane == 16, h, 0.0)) * inv_b
        kl = jnp.sum(jnp.where(lane == 17, h, 0.0)) * inv_b
        acc1_ref[...] = acc1.reshape(1, 1)
        ece_ref[...] = ece.reshape(1, 1)
        ent_ref[...] = ent.reshape(1, 1)
        kl_ref[...] = kl.reshape(1, 1)


def _tc_metrics(pred_t, target, bounds):
    one_spec = pl.BlockSpec((1, 1), lambda i: (0, 0))
    return pl.pallas_call(
        _tc_body,
        grid=(NBLK,),
        in_specs=[
            pl.BlockSpec((C, BBLK), lambda i: (0, i)),
            pl.BlockSpec((BBLK,), lambda i: (i,)),
            pl.BlockSpec((1, N_BINS + 1), lambda i: (0, 0)),
        ],
        out_specs=[one_spec, one_spec, one_spec, one_spec],
        out_shape=[jax.ShapeDtypeStruct((1, 1), jnp.float32)] * 4,
        scratch_shapes=[pltpu.VMEM((1, 128), jnp.float32)],
    )(pred_t, target, bounds)


def kernel(pred, target):
    boundaries = jnp.linspace(0.0, 1.0, N_BINS + 1).astype(jnp.float32)
    bounds = boundaries.reshape(1, N_BINS + 1)
    acc1, ece, ent, kl = _tc_metrics(pred.T, target, bounds)
    return (acc1.reshape(1), ece.reshape(1), ent[0, 0], kl[0, 0])


# speedup vs baseline: 12.9105x; 1.0577x over previous
import math

import jax
import jax.numpy as jnp
from jax import lax
from jax.experimental import pallas as pl
from jax.experimental.pallas import tpu as pltpu

B = 16384
C = 1000
N_BINS = 15
VIRTUAL_PROB = 0.9

BBLK = 2048
NBLK = B // BBLK

_Q = (1.0 - VIRTUAL_PROB) / (C - 1)
_KL_CONST = VIRTUAL_PROB * math.log(VIRTUAL_PROB) + (C - 1) * _Q * math.log(_Q)
_PMQ = VIRTUAL_PROB - _Q


def _tc_body(x_ref, tgt_ref, bnd_ref, acc1_ref, ece_ref, ent_ref, kl_ref,
             hist_ref):
    pid = pl.program_id(0)
    x = x_ref[...]
    tgt = tgt_ref[...]
    m = jnp.max(x, axis=0)
    e = jnp.exp(x - m[None, :])
    s = jnp.sum(e, axis=0)
    dot = jnp.sum(e * x, axis=0)
    sum_pred = jnp.sum(x, axis=0)
    rows = lax.broadcasted_iota(jnp.int32, (C, BBLK), 0)
    t_logit = jnp.sum(jnp.where(rows == tgt[None, :], x, 0.0), axis=0)
    lse = m + jnp.log(s)
    conf = 1.0 / s
    acc = (t_logit == m).astype(jnp.float32)
    ent_row = lse - dot / s
    kl_row = (jnp.float32(_KL_CONST)
              - jnp.float32(_Q) * (sum_pred - jnp.float32(C) * lse)
              - jnp.float32(_PMQ) * (t_logit - lse))

    cmp = (conf[:, None] > bnd_ref[...]).astype(jnp.int32)
    idx = jnp.sum(cmp[:, 1:N_BINS], axis=1)

    lanes = lax.broadcasted_iota(jnp.int32, (1, 128), 1)
    onehot = idx[:, None] == lanes
    diff = conf - acc
    contrib = jnp.sum(jnp.where(onehot, diff[:, None], 0.0), axis=0)
    scal = jnp.where(lanes[0] == 15, jnp.sum(acc),
                     jnp.where(lanes[0] == 16, jnp.sum(ent_row),
                               jnp.where(lanes[0] == 17, jnp.sum(kl_row),
                                         0.0)))
    upd = (contrib + scal)[None, :]

    @pl.when(pid == 0)
    def _():
        hist_ref[...] = upd

    @pl.when(pid > 0)
    def _():
        hist_ref[...] = hist_ref[...] + upd

    @pl.when(pid == NBLK - 1)
    def _():
        inv_b = 1.0 / B
        h = hist_ref[0, :]
        lane = lanes[0]
        ece = jnp.sum(jnp.where(lane < N_BINS, jnp.abs(h), 0.0)) * inv_b
        acc1 = jnp.sum(jnp.where(lane == 15, h, 0.0)) * inv_b
        ent = jnp.sum(jnp.where(lane == 16, h, 0.0)) * inv_b
        kl = jnp.sum(jnp.where(lane == 17, h, 0.0)) * inv_b
        acc1_ref[...] = acc1.reshape(1, 1)
        ece_ref[...] = ece.reshape(1, 1)
        ent_ref[...] = ent.reshape(1, 1)
        kl_ref[...] = kl.reshape(1, 1)


def _tc_metrics(pred_t, target, bounds):
    one_spec = pl.BlockSpec((1, 1), lambda i: (0, 0))
    return pl.pallas_call(
        _tc_body,
        grid=(NBLK,),
        in_specs=[
            pl.BlockSpec((C, BBLK), lambda i: (0, i)),
            pl.BlockSpec((BBLK,), lambda i: (i,)),
            pl.BlockSpec((1, N_BINS + 1), lambda i: (0, 0)),
        ],
        out_specs=[one_spec, one_spec, one_spec, one_spec],
        out_shape=[jax.ShapeDtypeStruct((1, 1), jnp.float32)] * 4,
        scratch_shapes=[pltpu.VMEM((1, 128), jnp.float32)],
    )(pred_t, target, bounds)


def kernel(pred, target):
    boundaries = jnp.linspace(0.0, 1.0, N_BINS + 1).astype(jnp.float32)
    bounds = boundaries.reshape(1, N_BINS + 1)
    acc1, ece, ent, kl = _tc_metrics(pred.T, target, bounds)
    return (acc1.reshape(1), ece.reshape(1), ent[0, 0], kl[0, 0])
